# baseline (speedup 1.0000x reference)
import functools

import jax
import jax.numpy as jnp
from jax import lax
from jax.experimental import pallas as pl
from jax.experimental.pallas import tpu as pltpu
from jax.experimental.pallas import tpu_sc as plsc

_NC = 2
_NS = 16
_NW = _NC * _NS
_L = 16
_RB = 32


def _sc_body(CW, RPW, x_hbm, t_hbm, xt_out, x0_out, rs_out,
             tgt_v, buf, xt_v, x0_v, rs_v, sem0, sem1):
    wid = lax.axis_index("s") * _NC + lax.axis_index("c")
    base = wid * RPW
    nbatch = RPW // _RB

    pltpu.sync_copy(t_hbm.at[pl.ds(base, RPW)], tgt_v)

    lane = lax.iota(jnp.int32, _L)
    zero = jnp.zeros((_L,), jnp.int32)
    ones = jnp.ones((_L,), jnp.float32)
    nch = CW // _L

    def issue(b, p, sem):
        pltpu.async_copy(
            x_hbm.at[pl.ds(base + b * _RB, _RB), pl.ds(0, CW)],
            buf.at[p], sem)

    def drain(p, sem):
        pltpu.make_async_copy(
            x_hbm.at[pl.ds(0, _RB), pl.ds(0, CW)], buf.at[p], sem).wait()

    def process(b, p):
        bv = buf.at[p]
        for c in range(_RB // _L):
            off = b * _RB + c * _L
            t = tgt_v[pl.ds(off, _L)]
            v_xt = jnp.zeros((_L,), jnp.float32)
            v_x0 = jnp.zeros((_L,), jnp.float32)
            v_rs = ones
            for l in range(_L):
                r = c * _L + l
                tv = t[l]
                tcl = jnp.minimum(tv, CW - 1)
                w16 = (tcl // _L) * _L
                win = bv[r, pl.ds(w16, _L)]
                xt_l = win.at[jnp.full((_L,), tcl % _L)].get(
                    mode="promise_in_bounds")
                x0_l = bv[r, pl.ds(0, _L)][0]

                def with_row(r=r):
                    def acc_chunk(k, a):
                        return a + jnp.exp(bv[r, pl.ds(k * _L, _L)])

                    acc = lax.fori_loop(0, nch, acc_chunk,
                                        jnp.zeros((_L,), jnp.float32))
                    s = acc[0]
                    for q in range(1, _L):
                        s = s + acc[q]
                    return s

                s = lax.cond(tv == 0, with_row, lambda: jnp.float32(1.0))
                m = lane == l
                v_xt = jnp.where(m, xt_l, v_xt)
                v_x0 = jnp.where(m, x0_l, v_x0)
                v_rs = jnp.where(m, s, v_rs)
            xt_v[pl.ds(off, _L)] = v_xt
            x0_v[pl.ds(off, _L)] = v_x0
            rs_v[pl.ds(off, _L)] = v_rs

    issue(0, 0, sem0)
    issue(1, 1, sem1)

    def pair(k, _):
        drain(0, sem0)
        process(2 * k, 0)

        @pl.when(k < nbatch // 2 - 1)
        def _():
            issue(2 * k + 2, 0, sem0)

        drain(1, sem1)
        process(2 * k + 1, 1)

        @pl.when(k < nbatch // 2 - 1)
        def _():
            issue(2 * k + 3, 1, sem1)

        return 0

    lax.fori_loop(0, nbatch // 2, pair, 0)

    pltpu.sync_copy(xt_v, xt_out.at[pl.ds(base, RPW)])
    pltpu.sync_copy(x0_v, x0_out.at[pl.ds(base, RPW)])
    pltpu.sync_copy(rs_v, rs_out.at[pl.ds(base, RPW)])


def _tail_body(CW, CT, x_ref, t_ref, ts_out, xt_out):
    x = x_ref[...]
    blk = x.shape[0]
    t = jnp.reshape(t_ref[0], (blk, 1))
    col = jax.lax.broadcasted_iota(jnp.int32, (blk, 128), 1)
    valid = col < CT
    e = jnp.where(valid, jnp.exp(x), 0.0)
    ts = jnp.sum(e, axis=1, keepdims=True)
    sel = jnp.where((col + CW) == t, x, 0.0)
    xt = jnp.sum(jnp.where(valid, sel, 0.0), axis=1, keepdims=True)
    ts_out[0] = jnp.reshape(ts, (1, blk))
    xt_out[0] = jnp.reshape(xt, (1, blk))


def _combine_body(B, CW, xt_ref, x0_ref, rs_ref, ts_ref, xtt_ref, t_ref,
                  out_ref):
    t = t_ref[...]
    xt = jnp.where(t < CW, xt_ref[...], xtt_ref[...])
    x0 = x0_ref[...]
    d = x0 - xt
    sp = jnp.maximum(d, 0.0) + jnp.log(1.0 + jnp.exp(-jnp.abs(d)))
    s = rs_ref[...] + ts_ref[...]
    lz = jnp.log(s) - x0
    loss_rows = jnp.where(t == 0, lz, sp)
    out_ref[0, 0] = jnp.sum(loss_rows) / B


def kernel(inputs, targets):
    B, C = inputs.shape
    CW = (C // 128) * 128
    CT = C - CW
    RPW = B // _NW
    t32 = targets.astype(jnp.int32)

    vec = jax.ShapeDtypeStruct((B,), jnp.float32)
    sc = pl.kernel(
        functools.partial(_sc_body, CW, RPW),
        out_type=(vec, vec, vec),
        mesh=plsc.VectorSubcoreMesh(core_axis_name="c", subcore_axis_name="s"),
        scratch_types=[
            pltpu.VMEM((RPW,), jnp.int32),
            pltpu.VMEM((2, _RB, CW), jnp.float32),
            pltpu.VMEM((RPW,), jnp.float32),
            pltpu.VMEM((RPW,), jnp.float32),
            pltpu.VMEM((RPW,), jnp.float32),
            pltpu.SemaphoreType.DMA,
            pltpu.SemaphoreType.DMA,
        ],
    )
    xt, x0, rs = sc(inputs, t32)

    BLK = 256
    grid = B // BLK
    t3 = t32.reshape(grid, 1, BLK)
    ts_tail, xt_tail = pl.pallas_call(
        functools.partial(_tail_body, CW, CT),
        grid=(grid,),
        in_specs=[
            pl.BlockSpec((BLK, 128), lambda i: (i, CW // 128)),
            pl.BlockSpec((1, 1, BLK), lambda i: (i, 0, 0)),
        ],
        out_specs=[
            pl.BlockSpec((1, 1, BLK), lambda i: (i, 0, 0)),
            pl.BlockSpec((1, 1, BLK), lambda i: (i, 0, 0)),
        ],
        out_shape=[
            jax.ShapeDtypeStruct((grid, 1, BLK), jnp.float32),
            jax.ShapeDtypeStruct((grid, 1, BLK), jnp.float32),
        ],
    )(inputs, t3)

    R = 128
    out = pl.pallas_call(
        functools.partial(_combine_body, B, CW),
        out_specs=pl.BlockSpec(memory_space=pltpu.SMEM),
        out_shape=jax.ShapeDtypeStruct((1, 1), jnp.float32),
    )(xt.reshape(R, B // R), x0.reshape(R, B // R), rs.reshape(R, B // R),
      ts_tail.reshape(R, B // R), xt_tail.reshape(R, B // R),
      t32.reshape(R, B // R))
    return out[0, 0]


# speedup vs baseline: 1.0695x; 1.0695x over previous
import jax
import jax.numpy as jnp
from jax.experimental import pallas as pl
from jax.experimental.pallas import tpu as pltpu

_BLK = 256
_NBUF = 6


def _body(x_hbm, t_ref, out_ref, buf, sem):
    pid = pl.program_id(0)
    n = pl.num_programs(0)

    @pl.when(pid == 0)
    def _():
        for i in range(_NBUF):
            pltpu.make_async_copy(
                x_hbm.at[pl.ds(i * _BLK, _BLK), :], buf.at[i], sem.at[i]
            ).start()

    slot = jax.lax.rem(pid, _NBUF)
    pltpu.make_async_copy(
        x_hbm.at[pl.ds(pid * _BLK, _BLK), :], buf.at[slot], sem.at[slot]
    ).wait()

    x = buf[slot]
    t = t_ref[0]
    blk, c = x.shape
    inv_b = 1.0 / (blk * n)

    col = jax.lax.broadcasted_iota(jnp.int32, (blk, c), 1)
    xt = jnp.sum(jnp.where(col == t, x, 0.0), axis=1, keepdims=True)
    x0 = x[:, 0:1]

    d = x0 - xt
    sp = jnp.maximum(d, 0.0) + jnp.log(1.0 + jnp.exp(-jnp.abs(d)))

    @pl.when(pid == 0)
    def _():
        out_ref[0, 0] = 0.0

    out_ref[0, 0] += jnp.sum(jnp.where(t == 0, 0.0, sp)) * inv_b

    @pl.when(jnp.min(t) == 0)
    def _():
        s = jnp.sum(jnp.exp(x), axis=1, keepdims=True)
        lz = jnp.log(s) - x0
        out_ref[0, 0] += jnp.sum(jnp.where(t == 0, lz, 0.0)) * inv_b

    @pl.when(pid + _NBUF < n)
    def _():
        pltpu.make_async_copy(
            x_hbm.at[pl.ds((pid + _NBUF) * _BLK, _BLK), :],
            buf.at[slot], sem.at[slot]
        ).start()


def kernel(inputs, targets):
    B, C = inputs.shape
    grid = B // _BLK
    t3 = targets.astype(jnp.int32).reshape(grid, _BLK, 1)

    out = pl.pallas_call(
        _body,
        grid=(grid,),
        in_specs=[
            pl.BlockSpec(memory_space=pl.ANY),
            pl.BlockSpec((1, _BLK, 1), lambda i: (i, 0, 0)),
        ],
        out_specs=pl.BlockSpec(memory_space=pltpu.SMEM),
        out_shape=jax.ShapeDtypeStruct((1, 1), jnp.float32),
        scratch_shapes=[
            pltpu.VMEM((_NBUF, _BLK, C), jnp.float32),
            pltpu.SemaphoreType.DMA((_NBUF,)),
        ],
    )(inputs, t3)
    return out[0, 0]


# speedup vs baseline: 1.2965x; 1.2122x over previous
import jax
import jax.numpy as jnp
from jax.experimental import pallas as pl
from jax.experimental.pallas import tpu as pltpu

_BLK = 512


def _body(zf_ref, x_ref, t_ref, out_ref):
    pid = pl.program_id(0)
    x = x_ref[...]
    t = t_ref[0]
    blk, c = x.shape
    inv_b = 1.0 / (blk * pl.num_programs(0))

    col = jax.lax.broadcasted_iota(jnp.int32, (blk, c), 1)
    xt = jnp.sum(jnp.where(col == t, x, 0.0), axis=1, keepdims=True)
    x0 = x[:, 0:1]

    d = x0 - xt
    sp = jnp.maximum(d, 0.0) + jnp.log(1.0 + jnp.exp(-jnp.abs(d)))

    @pl.when(pid == 0)
    def _():
        out_ref[0, 0] = 0.0

    out_ref[0, 0] += jnp.sum(sp) * inv_b

    @pl.when(zf_ref[pid] != 0)
    def _():
        s = jnp.sum(jnp.exp(x), axis=1, keepdims=True)
        lz = jnp.log(s) - x0
        out_ref[0, 0] += jnp.sum(jnp.where(t == 0, lz - sp, 0.0)) * inv_b


def kernel(inputs, targets):
    B, C = inputs.shape
    grid = B // _BLK
    t32 = targets.astype(jnp.int32)
    t3 = t32.reshape(grid, _BLK, 1)
    zflags = jnp.any(t3 == 0, axis=(1, 2)).astype(jnp.int32)

    out = pl.pallas_call(
        _body,
        grid=(grid,),
        in_specs=[
            pl.BlockSpec((B // _BLK,), lambda i: (0,), memory_space=pltpu.SMEM),
            pl.BlockSpec((_BLK, C), lambda i: (i, 0)),
            pl.BlockSpec((1, _BLK, 1), lambda i: (i, 0, 0)),
        ],
        out_specs=pl.BlockSpec(memory_space=pltpu.SMEM),
        out_shape=jax.ShapeDtypeStruct((1, 1), jnp.float32),
    )(zflags, inputs, t3)
    return out[0, 0]


# speedup vs baseline: 1.3888x; 1.0712x over previous
import jax
import jax.numpy as jnp
from jax.experimental import pallas as pl
from jax.experimental.pallas import tpu as pltpu

_BLK = 1024


def _body(zf_ref, x_ref, t_ref, out_ref):
    pid = pl.program_id(0)
    x = x_ref[...]
    t = t_ref[0]
    blk, c = x.shape
    inv_b = 1.0 / (blk * pl.num_programs(0))

    col = jax.lax.broadcasted_iota(jnp.int32, (blk, c), 1)
    xt = jnp.sum(jnp.where(col == t, x, 0.0), axis=1, keepdims=True)
    x0 = x[:, 0:1]

    d = x0 - xt
    sp = jnp.maximum(d, 0.0) + jnp.log(1.0 + jnp.exp(-jnp.abs(d)))

    @pl.when(pid == 0)
    def _():
        out_ref[0, 0] = 0.0

    out_ref[0, 0] += jnp.sum(sp) * inv_b

    @pl.when(zf_ref[pid] != 0)
    def _():
        s = jnp.sum(jnp.exp(x), axis=1, keepdims=True)
        lz = jnp.log(s) - x0
        out_ref[0, 0] += jnp.sum(jnp.where(t == 0, lz - sp, 0.0)) * inv_b


def kernel(inputs, targets):
    B, C = inputs.shape
    grid = B // _BLK
    t32 = targets.astype(jnp.int32)
    t3 = t32.reshape(grid, _BLK, 1)
    zflags = jnp.any(t3 == 0, axis=(1, 2)).astype(jnp.int32)

    out = pl.pallas_call(
        _body,
        grid=(grid,),
        in_specs=[
            pl.BlockSpec((B // _BLK,), lambda i: (0,), memory_space=pltpu.SMEM),
            pl.BlockSpec((_BLK, C), lambda i: (i, 0)),
            pl.BlockSpec((1, _BLK, 1), lambda i: (i, 0, 0)),
        ],
        out_specs=pl.BlockSpec(memory_space=pltpu.SMEM),
        out_shape=jax.ShapeDtypeStruct((1, 1), jnp.float32),
    )(zflags, inputs, t3)
    return out[0, 0]


# speedup vs baseline: 1.4389x; 1.0360x over previous
import jax
import jax.numpy as jnp
from jax.experimental import pallas as pl
from jax.experimental.pallas import tpu as pltpu

_BLK = 2048


def _body(zf_ref, x_ref, t_ref, out_ref):
    pid = pl.program_id(0)
    x = x_ref[...]
    t = t_ref[0]
    blk, c = x.shape
    inv_b = 1.0 / (blk * pl.num_programs(0))

    col = jax.lax.broadcasted_iota(jnp.int32, (blk, c), 1)
    xt = jnp.sum(jnp.where(col == t, x, 0.0), axis=1, keepdims=True)
    x0 = x[:, 0:1]

    d = x0 - xt
    sp = jnp.maximum(d, 0.0) + jnp.log(1.0 + jnp.exp(-jnp.abs(d)))

    @pl.when(pid == 0)
    def _():
        out_ref[0, 0] = 0.0

    out_ref[0, 0] += jnp.sum(sp) * inv_b

    @pl.when(zf_ref[pid] != 0)
    def _():
        s = jnp.sum(jnp.exp(x), axis=1, keepdims=True)
        lz = jnp.log(s) - x0
        out_ref[0, 0] += jnp.sum(jnp.where(t == 0, lz - sp, 0.0)) * inv_b


def kernel(inputs, targets):
    B, C = inputs.shape
    grid = B // _BLK
    t32 = targets.astype(jnp.int32)
    t3 = t32.reshape(grid, _BLK, 1)
    zflags = jnp.any(t3 == 0, axis=(1, 2)).astype(jnp.int32)

    out = pl.pallas_call(
        _body,
        grid=(grid,),
        in_specs=[
            pl.BlockSpec((B // _BLK,), lambda i: (0,), memory_space=pltpu.SMEM),
            pl.BlockSpec((_BLK, C), lambda i: (i, 0)),
            pl.BlockSpec((1, _BLK, 1), lambda i: (i, 0, 0)),
        ],
        out_specs=pl.BlockSpec(memory_space=pltpu.SMEM),
        out_shape=jax.ShapeDtypeStruct((1, 1), jnp.float32),
    )(zflags, inputs, t3)
    return out[0, 0]
